# baseline (speedup 1.0000x reference)
import functools
import math

import jax
import jax.numpy as jnp
from jax import lax
from jax.experimental import pallas as pl
from jax.experimental.pallas import tpu as pltpu
from jax.experimental.pallas import tpu_sc as plsc

_VOCAB = 1000000
_D = 64
_B = 1024
_L = 200
_N = _B * _L
_NC = 2
_NS = 16
_NW = _NC * _NS
_PER_W = _N // _NW
_CHUNK = 128
_J = _PER_W // _CHUNK
_NBUF = 5
_G = _J // _NBUF


def _scale_rows(src, dst):
  @plsc.parallel_loop(0, _CHUNK, 1, unroll=4)
  def _(r):
    for c in range(_D // 16):
      sl = pl.ds(c * 16, 16)
      dst[r, sl] = src[r, sl] * 8.0


def _body(x_hbm, lut_hbm, out_hbm, idx_v, rows_g, rows_s, gsems, ssems):
  wid = lax.axis_index("s") * _NC + lax.axis_index("c")

  pltpu.sync_copy(x_hbm.at[pl.ds(wid * _PER_W, _PER_W)], idx_v)

  out_base = wid * _PER_W

  def fire_gather(j, b):
    pltpu.make_async_copy(
        lut_hbm.at[idx_v.at[pl.ds(j * _CHUNK, _CHUNK)]], rows_g[b],
        gsems[b]).start()

  def wait_gather(b):
    pltpu.make_async_copy(
        lut_hbm.at[idx_v.at[pl.ds(0, _CHUNK)]], rows_g[b], gsems[b]).wait()

  def fire_store(j, b):
    d = pltpu.make_async_copy(
        rows_s[b],
        out_hbm.at[pl.ds(out_base + j * _CHUNK, _CHUNK)],
        ssems[b])
    d.start()
    return d

  def wait_store(b):
    pltpu.make_async_copy(
        rows_s[b],
        out_hbm.at[pl.ds(out_base, _CHUNK)],
        ssems[b]).wait()

  for b in range(_NBUF):
    fire_gather(b, b)

  for b in range(_NBUF):
    wait_gather(b)
    _scale_rows(rows_g[b], rows_s[b])
    fire_gather(b + _NBUF, b)
    fire_store(b, b)

  def group(g, _):
    for b in range(_NBUF):
      j = g * _NBUF + b
      wait_gather(b)
      wait_store(b)
      _scale_rows(rows_g[b], rows_s[b])
      fire_gather(j + _NBUF, b)
      fire_store(j, b)
    return _

  lax.fori_loop(1, _G - 1, group, 0, unroll=False)

  last = []
  for b in range(_NBUF):
    j = (_G - 1) * _NBUF + b
    wait_gather(b)
    wait_store(b)
    _scale_rows(rows_g[b], rows_s[b])
    last.append(fire_store(j, b))
  for d in last:
    d.wait()


@jax.jit
def kernel(x, lut):
  xf = x.T.astype(jnp.int32).reshape(_N)
  lutp = jnp.pad(lut, ((0, 0), (0, 16)))
  mesh = plsc.VectorSubcoreMesh(core_axis_name="c", subcore_axis_name="s")
  scratch = (
      pltpu.VMEM((_PER_W,), jnp.int32),
      [pltpu.VMEM((_CHUNK, _D + 16), jnp.float32)] * _NBUF,
      [pltpu.VMEM((_CHUNK, _D), jnp.float32)] * _NBUF,
      [pltpu.SemaphoreType.DMA] * _NBUF,
      [pltpu.SemaphoreType.DMA] * _NBUF,
  )
  mid = pl.kernel(
      _body,
      out_type=jax.ShapeDtypeStruct((_N, _D), jnp.float32),
      mesh=mesh,
      scratch_types=scratch,
      compiler_params=pltpu.CompilerParams(
          use_tc_tiling_on_sc=False,
          disable_bounds_checks=True,
      ),
  )(xf, lutp)
  return mid.reshape(_L, _B, _D).transpose(1, 0, 2)


# speedup vs baseline: 1.7446x; 1.7446x over previous
import jax
import jax.numpy as jnp
from jax import lax
from jax.experimental import pallas as pl
from jax.experimental.pallas import tpu as pltpu
from jax.experimental.pallas import tpu_sc as plsc

_VOCAB = 1000000
_D = 64
_B = 1024
_L = 200
_N = _B * _L
_NC = 2
_NS = 16
_NW = _NC * _NS
_PER_W = _N // _NW
_CHUNK = 128
_J = _PER_W // _CHUNK
_NBUF = 5
_G = _J // _NBUF


def _scale_rows(src, dst):
  @plsc.parallel_loop(0, _CHUNK, 1, unroll=4)
  def _(r):
    for c in range(_D // 16):
      sl = pl.ds(c * 16, 16)
      dst[r, sl] = src[r, sl] * 8.0


def _body(x_hbm, lut_hbm, out_hbm, idx_v, rows_g, rows_s, gsems, ssems):
  wid = lax.axis_index("s") * _NC + lax.axis_index("c")

  pltpu.sync_copy(x_hbm.at[pl.ds(wid * _PER_W, _PER_W)], idx_v)

  out_base = wid * _PER_W

  def fire_gather(j, b):
    pltpu.make_async_copy(
        lut_hbm.at[idx_v.at[pl.ds(j * _CHUNK, _CHUNK)]], rows_g[b],
        gsems[b]).start()

  def wait_gather(b):
    pltpu.make_async_copy(
        lut_hbm.at[idx_v.at[pl.ds(0, _CHUNK)]], rows_g[b], gsems[b]).wait()

  def fire_store(j, b):
    d = pltpu.make_async_copy(
        rows_s[b],
        out_hbm.at[pl.ds(out_base + j * _CHUNK, _CHUNK)],
        ssems[b])
    d.start()
    return d

  def wait_store(b):
    pltpu.make_async_copy(
        rows_s[b],
        out_hbm.at[pl.ds(out_base, _CHUNK)],
        ssems[b]).wait()

  for b in range(_NBUF):
    fire_gather(b, b)

  for b in range(_NBUF):
    wait_gather(b)
    _scale_rows(rows_g[b], rows_s[b])
    fire_gather(b + _NBUF, b)
    fire_store(b, b)

  def group(g, _):
    for b in range(_NBUF):
      j = g * _NBUF + b
      wait_gather(b)
      wait_store(b)
      _scale_rows(rows_g[b], rows_s[b])
      fire_gather(j + _NBUF, b)
      fire_store(j, b)
    return _

  lax.fori_loop(1, _G - 1, group, 0, unroll=False)

  last = []
  for b in range(_NBUF):
    j = (_G - 1) * _NBUF + b
    wait_gather(b)
    wait_store(b)
    _scale_rows(rows_g[b], rows_s[b])
    last.append(fire_store(j, b))
  for d in last:
    d.wait()


@jax.jit
def kernel(x, lut):
  xf = x.T.astype(jnp.int32).reshape(_N)
  lutp = jnp.pad(lut, ((0, 0), (0, _D)))
  mesh = plsc.VectorSubcoreMesh(core_axis_name="c", subcore_axis_name="s")
  scratch = (
      pltpu.VMEM((_PER_W,), jnp.int32),
      [pltpu.VMEM((_CHUNK, 2 * _D), jnp.float32)] * _NBUF,
      [pltpu.VMEM((_CHUNK, _D), jnp.float32)] * _NBUF,
      [pltpu.SemaphoreType.DMA] * _NBUF,
      [pltpu.SemaphoreType.DMA] * _NBUF,
  )
  mid = pl.kernel(
      _body,
      out_type=jax.ShapeDtypeStruct((_N, _D), jnp.float32),
      mesh=mesh,
      scratch_types=scratch,
      compiler_params=pltpu.CompilerParams(
          use_tc_tiling_on_sc=False,
          disable_bounds_checks=True,
      ),
  )(xf, lutp)
  return mid.reshape(_L, _B, _D).transpose(1, 0, 2)
